# baseline (speedup 1.0000x reference)
import functools

import jax
import jax.numpy as jnp
from jax import lax
from jax.experimental import pallas as pl
from jax.experimental.pallas import tpu as pltpu
from jax.experimental.pallas import tpu_sc as plsc

L = 2
N = 10000
E = 160000
D = 128

NC = 2
NS = 16
NW = NC * NS
CH = 128
NCHUNK = E // CH
NPAD = 10240
NPS = NPAD // NS


def _silu(x):
  return x * jax.nn.sigmoid(x)


def _ln_res(base, o, s, b):
  m = jnp.mean(o, axis=-1, keepdims=True)
  v = jnp.mean((o - m) ** 2, axis=-1, keepdims=True)
  return base + (o - m) * lax.rsqrt(v + 1e-5) * s + b


def _bdot(x, w_ref):
  return jnp.dot(x.astype(jnp.bfloat16), w_ref[...].astype(jnp.bfloat16),
                 preferred_element_type=jnp.float32)


def _pre_body(n_ref, w1s_ref, w1d_ref, b1_ref, as_ref, ad_ref):
  n = n_ref[...]
  as_ref[...] = _bdot(n, w1s_ref) + b1_ref[...]
  ad_ref[...] = _bdot(n, w1d_ref)


def _edge_body(e_ref, g_ref, w1e_ref, w2_ref, b2_ref, s_ref, b_ref,
               out_ref):
  e = e_ref[...]
  pre = _bdot(e, w1e_ref) + g_ref[...]
  o = _bdot(_silu(pre), w2_ref) + b2_ref[...]
  out_ref[...] = _ln_res(e, o, s_ref[...], b_ref[...])


def _node_mlp(p0_ref, p1_ref, n_ref, w1a_ref, w1n_ref, b1_ref, w2_ref,
              b2_ref, s_ref, b_ref):
  n = n_ref[...]
  agg = p0_ref[...] + p1_ref[...]
  pre = _bdot(agg, w1a_ref) + _bdot(n, w1n_ref) + b1_ref[...]
  o = _bdot(_silu(pre), w2_ref) + b2_ref[...]
  return _ln_res(n, o, s_ref[...], b_ref[...])


def _node_body(p0_ref, p1_ref, n_ref, w1a_ref, w1n_ref, b1_ref, w2_ref,
               b2_ref, s_ref, b_ref, out_ref):
  out_ref[...] = _node_mlp(p0_ref, p1_ref, n_ref, w1a_ref, w1n_ref, b1_ref,
                           w2_ref, b2_ref, s_ref, b_ref)


def _node_pre_body(p0_ref, p1_ref, n_ref, w1a_ref, w1n_ref, b1_ref, w2_ref,
                   b2_ref, s_ref, b_ref, wns_ref, wnd_ref, nb1_ref,
                   out_ref, as_ref, ad_ref):
  nn = _node_mlp(p0_ref, p1_ref, n_ref, w1a_ref, w1n_ref, b1_ref,
                 w2_ref, b2_ref, s_ref, b_ref)
  out_ref[...] = nn
  as_ref[...] = _bdot(nn, wns_ref) + nb1_ref[...]
  ad_ref[...] = _bdot(nn, wnd_ref)


def _row_spec(bn):
  return pl.BlockSpec((bn, D), lambda i: (i, 0))


def _row_spec_h(bn):
  return pl.BlockSpec((bn, D // 2), lambda i: (i, 0))


_W = pl.BlockSpec((D, D), lambda i: (0, 0))
_V = pl.BlockSpec((1, D), lambda i: (0, 0))


def _pre_call(n, w1s, w1d, b1):
  bn = 2000
  return pl.pallas_call(
      _pre_body,
      grid=(N // bn,),
      in_specs=[_row_spec(bn), _W, _W, _V],
      out_specs=[_row_spec(bn), _row_spec(bn)],
      out_shape=[jax.ShapeDtypeStruct((N, D), jnp.float32)] * 2,
  )(n, w1s, w1d, b1)


def _edge_call(e, g, w1e, w2, b2, ln_s, ln_b):
  be = 2000
  return pl.pallas_call(
      _edge_body,
      grid=(E // be,),
      in_specs=[_row_spec(be), _row_spec(be), _W, _W, _V, _V, _V],
      out_specs=_row_spec(be),
      out_shape=jax.ShapeDtypeStruct((E, D), jnp.float32),
  )(e, g, w1e, w2, b2, ln_s, ln_b)


def _node_call(p0, p1, n, w1a, w1n, b1, w2, b2, ln_s, ln_b):
  bn = 2000
  return pl.pallas_call(
      _node_body,
      grid=(N // bn,),
      in_specs=[_row_spec(bn), _row_spec(bn), _row_spec(bn), _W, _W, _V, _W,
                _V, _V, _V],
      out_specs=_row_spec(bn),
      out_shape=jax.ShapeDtypeStruct((N, D), jnp.float32),
  )(p0, p1, n, w1a, w1n, b1, w2, b2, ln_s, ln_b)


def _node_pre_call(p0, p1, n, w1a, w1n, b1, w2, b2, ln_s, ln_b,
                   wns, wnd, nb1):
  bn = 2000
  return pl.pallas_call(
      _node_pre_body,
      grid=(N // bn,),
      in_specs=[_row_spec(bn), _row_spec(bn), _row_spec(bn), _W, _W, _V, _W,
                _V, _V, _V, _W, _W, _V],
      out_specs=[_row_spec(bn), _row_spec(bn), _row_spec(bn)],
      out_shape=[jax.ShapeDtypeStruct((N, D), jnp.float32),
                 jax.ShapeDtypeStruct((N, D), jnp.float32),
                 jax.ShapeDtypeStruct((N, D), jnp.float32)],
  )(p0, p1, n, w1a, w1n, b1, w2, b2, ln_s, ln_b, wns, wnd, nb1)


_SC_MESH = plsc.VectorSubcoreMesh(
    core_axis_name="c", subcore_axis_name="s", num_cores=NC, num_subcores=NS)


EPT = E // NW
NCT = EPT // CH
TAIL = EPT - NCT * CH


def _gather_body(as_hbm, ad_hbm, src_hbm, dst_hbm, out_hbm,
                 isrc, idst, ra, rb, wo, sg, sw, st):
  wid = lax.axis_index("s") * NC + lax.axis_index("c")
  tb = pl.multiple_of(wid * EPT, 8)
  pltpu.sync_copy(src_hbm.at[pl.ds(tb, EPT)], isrc)
  pltpu.sync_copy(dst_hbm.at[pl.ds(tb, EPT)], idst)

  def fire(j, b):
    pltpu.async_copy(as_hbm.at[isrc.at[pl.ds(j * CH, CH)]], ra.at[b],
                     sg.at[b])
    pltpu.async_copy(ad_hbm.at[idst.at[pl.ds(j * CH, CH)]], rb.at[b],
                     sg.at[b])

  def consume(j, b):
    base = pl.multiple_of(tb + j * CH, 8)
    pltpu.make_async_copy(as_hbm.at[isrc.at[pl.ds(j * CH, CH)]], ra.at[b],
                          sg.at[b]).wait()
    pltpu.make_async_copy(ad_hbm.at[idst.at[pl.ds(j * CH, CH)]], rb.at[b],
                          sg.at[b]).wait()

    @plsc.parallel_loop(0, CH, unroll=4)
    def _add(r):
      for c in range(D // 16):
        wo[b, r, pl.ds(c * 16, 16)] = (ra[b, r, pl.ds(c * 16, 16)]
                                       + rb[b, r, pl.ds(c * 16, 16)])

    pltpu.async_copy(wo.at[b], out_hbm.at[pl.ds(base, CH)], sw.at[b])

  def drain_w(b):
    pltpu.make_async_copy(wo.at[b], out_hbm.at[pl.ds(tb, CH)],
                          sw.at[b]).wait()

  fire(0, 0)

  def body(j, carry):
    nb = (j + 1) % 2
    b = j % 2

    @pl.when(j + 1 < NCT)
    def _pref():
      @pl.when(j >= 1)
      def _():
        drain_w(nb)
      fire(j + 1, nb)

    consume(j, b)
    return carry

  lax.fori_loop(0, NCT, body, 0)
  jt = NCT * CH
  baset = pl.multiple_of(tb + jt, 8)
  pltpu.async_copy(as_hbm.at[isrc.at[pl.ds(jt, TAIL)]],
                   ra.at[0, pl.ds(0, TAIL)], st)
  pltpu.async_copy(ad_hbm.at[idst.at[pl.ds(jt, TAIL)]],
                   rb.at[0, pl.ds(0, TAIL)], st)
  drain_w(0)
  drain_w(1)
  pltpu.make_async_copy(as_hbm.at[isrc.at[pl.ds(jt, TAIL)]],
                        ra.at[0, pl.ds(0, TAIL)], st).wait()
  pltpu.make_async_copy(ad_hbm.at[idst.at[pl.ds(jt, TAIL)]],
                        rb.at[0, pl.ds(0, TAIL)], st).wait()

  @plsc.parallel_loop(0, TAIL)
  def _addt(r):
    for c in range(D // 16):
      wo[0, r, pl.ds(c * 16, 16)] = (ra[0, r, pl.ds(c * 16, 16)]
                                     + rb[0, r, pl.ds(c * 16, 16)])

  pltpu.sync_copy(wo.at[0, pl.ds(0, TAIL)], out_hbm.at[pl.ds(baset, TAIL)])


_gather_call = pl.kernel(
    _gather_body,
    out_type=jax.ShapeDtypeStruct((E, D), jnp.float32),
    mesh=_SC_MESH,
    scratch_types=[
        pltpu.VMEM((EPT,), jnp.int32),
        pltpu.VMEM((EPT,), jnp.int32),
        pltpu.VMEM((2, CH, D), jnp.float32),
        pltpu.VMEM((2, CH, D), jnp.float32),
        pltpu.VMEM((2, CH, D), jnp.float32),
        pltpu.SemaphoreType.DMA((2,)),
        pltpu.SemaphoreType.DMA((2,)),
        pltpu.SemaphoreType.DMA,
    ],
)


def _scatter_body(e_hbm, dst_hbm, zero_hbm, out_hbm,
                  didx, tidx, rows, accum, si, ss, st):
  c = lax.axis_index("c")
  s = lax.axis_index("s")
  wid = s * NC + c
  tb = pl.multiple_of(wid * EPT, 8)
  pltpu.sync_copy(zero_hbm.at[pl.ds(s * NPS, NPS)],
                  accum.at[pl.ds(s * NPS, NPS)])
  plsc.subcore_barrier()

  def fire_in(j, b):
    base = pl.multiple_of(tb + j * CH, 8)
    pltpu.async_copy(dst_hbm.at[pl.ds(base, CH)], didx.at[b], si.at[b])
    pltpu.async_copy(e_hbm.at[pl.ds(base, CH)], rows.at[b], si.at[b])

  def wait_in(j, b):
    base = pl.multiple_of(tb + j * CH, 8)
    pltpu.make_async_copy(dst_hbm.at[pl.ds(base, CH)], didx.at[b],
                          si.at[b]).wait()
    pltpu.make_async_copy(e_hbm.at[pl.ds(base, CH)], rows.at[b],
                          si.at[b]).wait()

  def drain_sc(b):
    pltpu.make_async_copy(rows.at[b], accum.at[didx.at[b]], ss.at[b]).wait()

  fire_in(0, 0)

  def body(j, carry):
    nb = (j + 1) % 2
    b = j % 2

    @pl.when(j + 1 < NCT)
    def _pref():
      @pl.when(j >= 1)
      def _():
        drain_sc(nb)
      fire_in(j + 1, nb)

    wait_in(j, b)
    pltpu.async_copy(rows.at[b], accum.at[didx.at[b]], ss.at[b], add=True)
    return carry

  lax.fori_loop(0, NCT, body, 0)
  baset = pl.multiple_of(tb + NCT * CH, 8)
  drain_sc(0)
  drain_sc(1)
  pltpu.sync_copy(dst_hbm.at[pl.ds(baset, TAIL)], tidx)
  pltpu.sync_copy(e_hbm.at[pl.ds(baset, TAIL)], rows.at[0, pl.ds(0, TAIL)])
  pltpu.sync_copy(rows.at[0, pl.ds(0, TAIL)], accum.at[tidx], add=True)
  plsc.subcore_barrier()
  pltpu.sync_copy(accum.at[pl.ds(s * NPS, NPS)],
                  out_hbm.at[c, pl.ds(s * NPS, NPS)])


_scatter_call = pl.kernel(
    _scatter_body,
    out_type=jax.ShapeDtypeStruct((NC, NPAD, D), jnp.float32),
    mesh=_SC_MESH,
    scratch_types=[
        pltpu.VMEM((2, CH), jnp.int32),
        pltpu.VMEM((TAIL,), jnp.int32),
        pltpu.VMEM((2, CH, D), jnp.float32),
        pltpu.VMEM_SHARED((NPAD, D), jnp.float32),
        pltpu.SemaphoreType.DMA((2,)),
        pltpu.SemaphoreType.DMA((2,)),
        pltpu.SemaphoreType.DMA,
    ],
)


def kernel(efeat, nfeat, edge_index, edge_w1, edge_b1, edge_w2, edge_b2,
           edge_ln_s, edge_ln_b, node_w1, node_b1, node_w2, node_b2,
           node_ln_s, node_ln_b):
  src = edge_index[0]
  dst = edge_index[1]
  zero = jnp.zeros((NPAD, D), jnp.float32)
  e, n = efeat, nfeat
  a_s, a_d = _pre_call(n, edge_w1[0, D:2 * D], edge_w1[0, 2 * D:],
                       edge_b1[0].reshape(1, D))
  for i in range(L):
    g = _gather_call(a_s, a_d, src, dst)
    e = _edge_call(e, g, edge_w1[i, :D], edge_w2[i],
                   edge_b2[i].reshape(1, D),
                   edge_ln_s[i].reshape(1, D), edge_ln_b[i].reshape(1, D))
    parts = _scatter_call(e, dst, zero)
    node_args = (parts[0], parts[1], n, node_w1[i, :D], node_w1[i, D:],
                 node_b1[i].reshape(1, D), node_w2[i],
                 node_b2[i].reshape(1, D), node_ln_s[i].reshape(1, D),
                 node_ln_b[i].reshape(1, D))
    if i + 1 < L:
      n, a_s, a_d = _node_pre_call(*node_args, edge_w1[i + 1, D:2 * D],
                                   edge_w1[i + 1, 2 * D:],
                                   edge_b1[i + 1].reshape(1, D))
    else:
      n = _node_call(*node_args)
  return (e, n)


# speedup vs baseline: 1.1044x; 1.1044x over previous
import functools

import numpy as np

import jax
import jax.numpy as jnp
from jax import lax
from jax.experimental import pallas as pl
from jax.experimental.pallas import tpu as pltpu
from jax.experimental.pallas import tpu_sc as plsc

L = 2
N = 10000
E = 160000
D = 128

NC = 2
NS = 16
NW = NC * NS
CH = 128
NCHUNK = E // CH
NPAD = 10240
NPS = NPAD // NS


def _silu(x):
  return x * jax.nn.sigmoid(x)


def _ln_res(base, o, s, b):
  m = jnp.mean(o, axis=-1, keepdims=True)
  v = jnp.mean((o - m) ** 2, axis=-1, keepdims=True)
  return base + (o - m) * lax.rsqrt(v + 1e-5) * s + b


def _bdot(x, w_ref):
  return jnp.dot(x.astype(jnp.bfloat16), w_ref[...].astype(jnp.bfloat16),
                 preferred_element_type=jnp.float32)


def _pre_body(n_ref, w1s_ref, w1d_ref, b1_ref, as_ref, ad_ref):
  n = n_ref[...]
  as_ref[...] = _bdot(n, w1s_ref) + b1_ref[...]
  ad_ref[...] = _bdot(n, w1d_ref)


def _edge_body(e_ref, g_ref, w1e_ref, w2_ref, b2_ref, s_ref, b_ref,
               out_ref):
  e = e_ref[...]
  pre = _bdot(e, w1e_ref) + g_ref[...].astype(jnp.float32)
  o = _bdot(_silu(pre), w2_ref) + b2_ref[...]
  out_ref[...] = _ln_res(e, o, s_ref[...], b_ref[...])


def _node_mlp(p0_ref, p1_ref, n_ref, w1a_ref, w1n_ref, b1_ref, w2_ref,
              b2_ref, s_ref, b_ref):
  n = n_ref[...]
  agg = p0_ref[...] + p1_ref[...]
  pre = _bdot(agg, w1a_ref) + _bdot(n, w1n_ref) + b1_ref[...]
  o = _bdot(_silu(pre), w2_ref) + b2_ref[...]
  return _ln_res(n, o, s_ref[...], b_ref[...])


def _node_body(p0_ref, p1_ref, n_ref, w1a_ref, w1n_ref, b1_ref, w2_ref,
               b2_ref, s_ref, b_ref, out_ref):
  out_ref[...] = _node_mlp(p0_ref, p1_ref, n_ref, w1a_ref, w1n_ref, b1_ref,
                           w2_ref, b2_ref, s_ref, b_ref)


def _node_pre_body(p0_ref, p1_ref, n_ref, w1a_ref, w1n_ref, b1_ref, w2_ref,
                   b2_ref, s_ref, b_ref, wns_ref, wnd_ref, nb1_ref,
                   out_ref, as_ref, ad_ref):
  nn = _node_mlp(p0_ref, p1_ref, n_ref, w1a_ref, w1n_ref, b1_ref,
                 w2_ref, b2_ref, s_ref, b_ref)
  out_ref[...] = nn
  as_ref[...] = _bdot(nn, wns_ref) + nb1_ref[...]
  ad_ref[...] = _bdot(nn, wnd_ref)


def _row_spec(bn):
  return pl.BlockSpec((bn, D), lambda i: (i, 0))


def _row_spec_h(bn):
  return pl.BlockSpec((bn, D // 2), lambda i: (i, 0))


_W = pl.BlockSpec((D, D), lambda i: (0, 0))
_V = pl.BlockSpec((1, D), lambda i: (0, 0))


def _pre_call(n, w1s, w1d, b1):
  bn = 2000
  return pl.pallas_call(
      _pre_body,
      grid=(N // bn,),
      in_specs=[_row_spec(bn), _W, _W, _V],
      out_specs=[_row_spec(bn), _row_spec(bn)],
      out_shape=[jax.ShapeDtypeStruct((N, D), jnp.float32)] * 2,
  )(n, w1s, w1d, b1)


def _edge_call(e, g, w1e, w2, b2, ln_s, ln_b):
  be = 4000
  return pl.pallas_call(
      _edge_body,
      grid=(E // be,),
      in_specs=[_row_spec(be), _row_spec(be), _W, _W, _V, _V, _V],
      out_specs=_row_spec(be),
      out_shape=jax.ShapeDtypeStruct((E, D), jnp.float32),
  )(e, g, w1e, w2, b2, ln_s, ln_b)


def _node_call(p0, p1, n, w1a, w1n, b1, w2, b2, ln_s, ln_b):
  bn = 2000
  return pl.pallas_call(
      _node_body,
      grid=(N // bn,),
      in_specs=[_row_spec(bn), _row_spec(bn), _row_spec(bn), _W, _W, _V, _W,
                _V, _V, _V],
      out_specs=_row_spec(bn),
      out_shape=jax.ShapeDtypeStruct((N, D), jnp.float32),
  )(p0, p1, n, w1a, w1n, b1, w2, b2, ln_s, ln_b)


def _node_pre_call(p0, p1, n, w1a, w1n, b1, w2, b2, ln_s, ln_b,
                   wns, wnd, nb1):
  bn = 2000
  return pl.pallas_call(
      _node_pre_body,
      grid=(N // bn,),
      in_specs=[_row_spec(bn), _row_spec(bn), _row_spec(bn), _W, _W, _V, _W,
                _V, _V, _V, _W, _W, _V],
      out_specs=[_row_spec(bn), _row_spec(bn), _row_spec(bn)],
      out_shape=[jax.ShapeDtypeStruct((N, D), jnp.float32),
                 jax.ShapeDtypeStruct((N, D), jnp.float32),
                 jax.ShapeDtypeStruct((N, D), jnp.float32)],
  )(p0, p1, n, w1a, w1n, b1, w2, b2, ln_s, ln_b, wns, wnd, nb1)


_SC_MESH = plsc.VectorSubcoreMesh(
    core_axis_name="c", subcore_axis_name="s", num_cores=NC, num_subcores=NS)


EPT = E // NW
NCT = EPT // CH
TAIL = EPT - NCT * CH


def _gather_body(as_hbm, ad_hbm, src_hbm, dst_hbm, out_hbm,
                 isrc, idst, ra, rb, wo, sg, sw, st):
  wid = lax.axis_index("s") * NC + lax.axis_index("c")
  tb = pl.multiple_of(wid * EPT, 8)
  pltpu.sync_copy(src_hbm.at[pl.ds(tb, EPT)], isrc)
  pltpu.sync_copy(dst_hbm.at[pl.ds(tb, EPT)], idst)

  def fire(j, b):
    pltpu.async_copy(as_hbm.at[isrc.at[pl.ds(j * CH, CH)]], ra.at[b],
                     sg.at[b])
    pltpu.async_copy(ad_hbm.at[idst.at[pl.ds(j * CH, CH)]], rb.at[b],
                     sg.at[b])

  def consume(j, b):
    base = pl.multiple_of(tb + j * CH, 8)
    pltpu.make_async_copy(as_hbm.at[isrc.at[pl.ds(j * CH, CH)]], ra.at[b],
                          sg.at[b]).wait()
    pltpu.make_async_copy(ad_hbm.at[idst.at[pl.ds(j * CH, CH)]], rb.at[b],
                          sg.at[b]).wait()

    @plsc.parallel_loop(0, CH, unroll=4)
    def _add(r):
      for c in range(D // 16):
        wo[b, r, pl.ds(c * 16, 16)] = (ra[b, r, pl.ds(c * 16, 16)]
                                       + rb[b, r, pl.ds(c * 16, 16)])

    pltpu.async_copy(wo.at[b], out_hbm.at[pl.ds(base, CH)], sw.at[b])

  def drain_w(b):
    pltpu.make_async_copy(wo.at[b], out_hbm.at[pl.ds(tb, CH)],
                          sw.at[b]).wait()

  fire(0, 0)

  def body(j, carry):
    nb = (j + 1) % 2
    b = j % 2

    @pl.when(j + 1 < NCT)
    def _pref():
      @pl.when(j >= 1)
      def _():
        drain_w(nb)
      fire(j + 1, nb)

    consume(j, b)
    return carry

  lax.fori_loop(0, NCT, body, 0)
  jt = NCT * CH
  baset = pl.multiple_of(tb + jt, 8)
  pltpu.async_copy(as_hbm.at[isrc.at[pl.ds(jt, TAIL)]],
                   ra.at[0, pl.ds(0, TAIL)], st)
  pltpu.async_copy(ad_hbm.at[idst.at[pl.ds(jt, TAIL)]],
                   rb.at[0, pl.ds(0, TAIL)], st)
  drain_w(0)
  drain_w(1)
  pltpu.make_async_copy(as_hbm.at[isrc.at[pl.ds(jt, TAIL)]],
                        ra.at[0, pl.ds(0, TAIL)], st).wait()
  pltpu.make_async_copy(ad_hbm.at[idst.at[pl.ds(jt, TAIL)]],
                        rb.at[0, pl.ds(0, TAIL)], st).wait()

  @plsc.parallel_loop(0, TAIL)
  def _addt(r):
    for c in range(D // 16):
      wo[0, r, pl.ds(c * 16, 16)] = (ra[0, r, pl.ds(c * 16, 16)]
                                     + rb[0, r, pl.ds(c * 16, 16)])

  pltpu.sync_copy(wo.at[0, pl.ds(0, TAIL)], out_hbm.at[pl.ds(baset, TAIL)])


_gather_call = pl.kernel(
    _gather_body,
    out_type=jax.ShapeDtypeStruct((E, D), jnp.float32),
    mesh=_SC_MESH,
    scratch_types=[
        pltpu.VMEM((EPT,), jnp.int32),
        pltpu.VMEM((EPT,), jnp.int32),
        pltpu.VMEM((2, CH, D), jnp.float32),
        pltpu.VMEM((2, CH, D), jnp.float32),
        pltpu.VMEM((2, CH, D), jnp.float32),
        pltpu.SemaphoreType.DMA((2,)),
        pltpu.SemaphoreType.DMA((2,)),
        pltpu.SemaphoreType.DMA,
    ],
)


def _scatter_body(e_hbm, dst_hbm, zero_hbm, out_hbm,
                  didx, tidx, rows, accum, si, ss, st):
  c = lax.axis_index("c")
  s = lax.axis_index("s")
  wid = s * NC + c
  tb = pl.multiple_of(wid * EPT, 8)
  pltpu.sync_copy(zero_hbm.at[pl.ds(s * NPS, NPS)],
                  accum.at[pl.ds(s * NPS, NPS)])
  plsc.subcore_barrier()

  def fire_in(j, b):
    base = pl.multiple_of(tb + j * CH, 8)
    pltpu.async_copy(dst_hbm.at[pl.ds(base, CH)], didx.at[b], si.at[b])
    pltpu.async_copy(e_hbm.at[pl.ds(base, CH)], rows.at[b], si.at[b])

  def wait_in(j, b):
    base = pl.multiple_of(tb + j * CH, 8)
    pltpu.make_async_copy(dst_hbm.at[pl.ds(base, CH)], didx.at[b],
                          si.at[b]).wait()
    pltpu.make_async_copy(e_hbm.at[pl.ds(base, CH)], rows.at[b],
                          si.at[b]).wait()

  def drain_sc(b):
    pltpu.make_async_copy(rows.at[b], accum.at[didx.at[b]], ss.at[b]).wait()

  fire_in(0, 0)

  def body(j, carry):
    nb = (j + 1) % 2
    b = j % 2

    @pl.when(j + 1 < NCT)
    def _pref():
      @pl.when(j >= 1)
      def _():
        drain_sc(nb)
      fire_in(j + 1, nb)

    wait_in(j, b)
    pltpu.async_copy(rows.at[b], accum.at[didx.at[b]], ss.at[b], add=True)
    return carry

  lax.fori_loop(0, NCT, body, 0)
  baset = pl.multiple_of(tb + NCT * CH, 8)
  drain_sc(0)
  drain_sc(1)
  pltpu.sync_copy(dst_hbm.at[pl.ds(baset, TAIL)], tidx)
  pltpu.sync_copy(e_hbm.at[pl.ds(baset, TAIL)], rows.at[0, pl.ds(0, TAIL)])
  pltpu.sync_copy(rows.at[0, pl.ds(0, TAIL)], accum.at[tidx], add=True)
  plsc.subcore_barrier()
  pltpu.sync_copy(accum.at[pl.ds(s * NPS, NPS)],
                  out_hbm.at[c, pl.ds(s * NPS, NPS)])


_scatter_call = pl.kernel(
    _scatter_body,
    out_type=jax.ShapeDtypeStruct((NC, NPAD, D), jnp.float32),
    mesh=_SC_MESH,
    scratch_types=[
        pltpu.VMEM((2, CH), jnp.int32),
        pltpu.VMEM((TAIL,), jnp.int32),
        pltpu.VMEM((2, CH, D), jnp.float32),
        pltpu.VMEM_SHARED((NPAD, D), jnp.float32),
        pltpu.SemaphoreType.DMA((2,)),
        pltpu.SemaphoreType.DMA((2,)),
        pltpu.SemaphoreType.DMA,
    ],
)


def kernel(efeat, nfeat, edge_index, edge_w1, edge_b1, edge_w2, edge_b2,
           edge_ln_s, edge_ln_b, node_w1, node_b1, node_w2, node_b2,
           node_ln_s, node_ln_b):
  src = edge_index[0]
  dst = edge_index[1]
  zero = jnp.zeros((NPAD, D), jnp.float32)
  w1s = edge_w1[:, D:2 * D]
  w1d = edge_w1[:, 2 * D:]
  b1 = edge_b1
  e, n = efeat, nfeat
  a_s, a_d = _pre_call(n, w1s[0], w1d[0], b1[0].reshape(1, D))
  for i in range(L):
    g = _gather_call(a_s, a_d, src, dst)
    e = _edge_call(e, g, edge_w1[i, :D], edge_w2[i],
                   edge_b2[i].reshape(1, D),
                   edge_ln_s[i].reshape(1, D), edge_ln_b[i].reshape(1, D))
    parts = _scatter_call(e, dst, zero)
    node_args = (parts[0], parts[1], n, node_w1[i, :D], node_w1[i, D:],
                 node_b1[i].reshape(1, D), node_w2[i],
                 node_b2[i].reshape(1, D), node_ln_s[i].reshape(1, D),
                 node_ln_b[i].reshape(1, D))
    if i + 1 < L:
      n, a_s, a_d = _node_pre_call(*node_args, w1s[i + 1], w1d[i + 1],
                                   b1[i + 1].reshape(1, D))
    else:
      n = _node_call(*node_args)
  return (e, n)


# speedup vs baseline: 1.1863x; 1.0742x over previous
import functools

import numpy as np

import jax
import jax.numpy as jnp
from jax import lax
from jax.experimental import pallas as pl
from jax.experimental.pallas import tpu as pltpu
from jax.experimental.pallas import tpu_sc as plsc

L = 2
N = 10000
E = 160000
D = 128

NC = 2
NS = 16
NW = NC * NS
CH = 128
NCHUNK = E // CH
NPAD = 10240
NPS = NPAD // NS


def _silu(x):
  return x * jax.nn.sigmoid(x)


def _ln_res(base, o, s, b):
  m = jnp.mean(o, axis=-1, keepdims=True)
  v = jnp.mean((o - m) ** 2, axis=-1, keepdims=True)
  return base + (o - m) * lax.rsqrt(v + 1e-5) * s + b


def _bdot(x, w):
  return jnp.dot(x.astype(jnp.bfloat16), w.astype(jnp.bfloat16),
                 preferred_element_type=jnp.float32)


def _pre_body(li, n_ref, w1_ref, b1_ref, as_ref, ad_ref):
  n = n_ref[...]
  as_ref[...] = _bdot(n, w1_ref[0, D:2 * D]) + b1_ref[li]
  ad_ref[...] = _bdot(n, w1_ref[0, 2 * D:])


def _edge_body(li, e_ref, g_ref, w1_ref, w2_ref, b2_ref, s_ref, b_ref,
               out_ref):
  e = e_ref[...]
  pre = _bdot(e, w1_ref[0, :D]) + g_ref[...]
  o = _bdot(_silu(pre), w2_ref[0]) + b2_ref[li]
  out_ref[...] = _ln_res(e, o, s_ref[li], b_ref[li])


def _node_mlp(li, p_ref, n_ref, w1_ref, b1_ref, w2_ref,
              b2_ref, s_ref, b_ref):
  n = n_ref[...]
  agg = p_ref[0] + p_ref[1]
  pre = _bdot(agg, w1_ref[0, :D]) + _bdot(n, w1_ref[0, D:]) + b1_ref[li]
  o = _bdot(_silu(pre), w2_ref[0]) + b2_ref[li]
  return _ln_res(n, o, s_ref[li], b_ref[li])


def _node_body(li, p_ref, n_ref, w1_ref, b1_ref, w2_ref,
               b2_ref, s_ref, b_ref, out_ref):
  out_ref[...] = _node_mlp(li, p_ref, n_ref, w1_ref, b1_ref,
                           w2_ref, b2_ref, s_ref, b_ref)


def _node_pre_body(li, p_ref, n_ref, w1_ref, b1_ref, w2_ref,
                   b2_ref, s_ref, b_ref, ew1_ref, eb1_ref,
                   out_ref, as_ref, ad_ref):
  nn = _node_mlp(li, p_ref, n_ref, w1_ref, b1_ref,
                 w2_ref, b2_ref, s_ref, b_ref)
  out_ref[...] = nn
  as_ref[...] = _bdot(nn, ew1_ref[0, D:2 * D]) + eb1_ref[li + 1]
  ad_ref[...] = _bdot(nn, ew1_ref[0, 2 * D:])


def _row_spec(bn):
  return pl.BlockSpec((bn, D), lambda i: (i, 0))


def _part_spec(bn):
  return pl.BlockSpec((NC, bn, D), lambda i: (0, i, 0))


def _lw_spec(li, rows):
  return pl.BlockSpec((1, rows, D), lambda i: (li, 0, 0))


_LV = pl.BlockSpec((L, D), lambda i: (0, 0))


def _pre_call(n, edge_w1, edge_b1, li):
  bn = 2000
  return pl.pallas_call(
      functools.partial(_pre_body, li),
      grid=(N // bn,),
      in_specs=[_row_spec(bn), _lw_spec(li, 3 * D), _LV],
      out_specs=[_row_spec(bn), _row_spec(bn)],
      out_shape=[jax.ShapeDtypeStruct((N, D), jnp.float32)] * 2,
  )(n, edge_w1, edge_b1)


def _edge_call(e, g, edge_w1, edge_w2, edge_b2, edge_ln_s, edge_ln_b, li):
  be = 8000
  return pl.pallas_call(
      functools.partial(_edge_body, li),
      grid=(E // be,),
      in_specs=[_row_spec(be), _row_spec(be), _lw_spec(li, 3 * D),
                _lw_spec(li, D), _LV, _LV, _LV],
      out_specs=_row_spec(be),
      out_shape=jax.ShapeDtypeStruct((E, D), jnp.float32),
  )(e, g, edge_w1, edge_w2, edge_b2, edge_ln_s, edge_ln_b)


def _node_args(parts, n, node_w1, node_b1, node_w2, node_b2, node_ln_s,
               node_ln_b, li, bn):
  specs = [_part_spec(bn), _row_spec(bn),
           _lw_spec(li, 2 * D), _LV, _lw_spec(li, D), _LV, _LV, _LV]
  args = (parts, n, node_w1, node_b1, node_w2, node_b2, node_ln_s,
          node_ln_b)
  return specs, args


def _node_call(parts, n, node_w1, node_b1, node_w2, node_b2, node_ln_s,
               node_ln_b, li):
  bn = 2000
  specs, args = _node_args(parts, n, node_w1, node_b1, node_w2, node_b2,
                           node_ln_s, node_ln_b, li, bn)
  return pl.pallas_call(
      functools.partial(_node_body, li),
      grid=(N // bn,),
      in_specs=specs,
      out_specs=_row_spec(bn),
      out_shape=jax.ShapeDtypeStruct((N, D), jnp.float32),
  )(*args)


def _node_pre_call(parts, n, node_w1, node_b1, node_w2, node_b2, node_ln_s,
                   node_ln_b, li, edge_w1, edge_b1):
  bn = 2000
  specs, args = _node_args(parts, n, node_w1, node_b1, node_w2, node_b2,
                           node_ln_s, node_ln_b, li, bn)
  return pl.pallas_call(
      functools.partial(_node_pre_body, li),
      grid=(N // bn,),
      in_specs=specs + [_lw_spec(li + 1, 3 * D), _LV],
      out_specs=[_row_spec(bn), _row_spec(bn), _row_spec(bn)],
      out_shape=[jax.ShapeDtypeStruct((N, D), jnp.float32),
                 jax.ShapeDtypeStruct((N, D), jnp.float32),
                 jax.ShapeDtypeStruct((N, D), jnp.float32)],
  )(*args, edge_w1, edge_b1)


_SC_MESH = plsc.VectorSubcoreMesh(
    core_axis_name="c", subcore_axis_name="s", num_cores=NC, num_subcores=NS)


EPT = E // NW
NCT = EPT // CH
TAIL = EPT - NCT * CH


def _gather_body(as_hbm, ad_hbm, src_hbm, dst_hbm, out_hbm,
                 isrc, idst, ra, rb, wo, sg, sw, st):
  wid = lax.axis_index("s") * NC + lax.axis_index("c")
  tb = pl.multiple_of(wid * EPT, 8)
  pltpu.sync_copy(src_hbm.at[pl.ds(tb, EPT)], isrc)
  pltpu.sync_copy(dst_hbm.at[pl.ds(tb, EPT)], idst)

  def fire(j, b):
    pltpu.async_copy(as_hbm.at[isrc.at[pl.ds(j * CH, CH)]], ra.at[b],
                     sg.at[b])
    pltpu.async_copy(ad_hbm.at[idst.at[pl.ds(j * CH, CH)]], rb.at[b],
                     sg.at[b])

  def consume(j, b):
    base = pl.multiple_of(tb + j * CH, 8)
    pltpu.make_async_copy(as_hbm.at[isrc.at[pl.ds(j * CH, CH)]], ra.at[b],
                          sg.at[b]).wait()
    pltpu.make_async_copy(ad_hbm.at[idst.at[pl.ds(j * CH, CH)]], rb.at[b],
                          sg.at[b]).wait()

    @plsc.parallel_loop(0, CH, unroll=4)
    def _add(r):
      for c in range(D // 16):
        wo[b, r, pl.ds(c * 16, 16)] = (ra[b, r, pl.ds(c * 16, 16)]
                                       + rb[b, r, pl.ds(c * 16, 16)])

    pltpu.async_copy(wo.at[b], out_hbm.at[pl.ds(base, CH)], sw.at[b])

  def drain_w(b):
    pltpu.make_async_copy(wo.at[b], out_hbm.at[pl.ds(tb, CH)],
                          sw.at[b]).wait()

  fire(0, 0)

  def body(j, carry):
    nb = (j + 1) % 2
    b = j % 2

    @pl.when(j + 1 < NCT)
    def _pref():
      @pl.when(j >= 1)
      def _():
        drain_w(nb)
      fire(j + 1, nb)

    consume(j, b)
    return carry

  lax.fori_loop(0, NCT, body, 0)
  jt = NCT * CH
  baset = pl.multiple_of(tb + jt, 8)
  pltpu.async_copy(as_hbm.at[isrc.at[pl.ds(jt, TAIL)]],
                   ra.at[0, pl.ds(0, TAIL)], st)
  pltpu.async_copy(ad_hbm.at[idst.at[pl.ds(jt, TAIL)]],
                   rb.at[0, pl.ds(0, TAIL)], st)
  drain_w(0)
  drain_w(1)
  pltpu.make_async_copy(as_hbm.at[isrc.at[pl.ds(jt, TAIL)]],
                        ra.at[0, pl.ds(0, TAIL)], st).wait()
  pltpu.make_async_copy(ad_hbm.at[idst.at[pl.ds(jt, TAIL)]],
                        rb.at[0, pl.ds(0, TAIL)], st).wait()

  @plsc.parallel_loop(0, TAIL)
  def _addt(r):
    for c in range(D // 16):
      wo[0, r, pl.ds(c * 16, 16)] = (ra[0, r, pl.ds(c * 16, 16)]
                                     + rb[0, r, pl.ds(c * 16, 16)])

  pltpu.sync_copy(wo.at[0, pl.ds(0, TAIL)], out_hbm.at[pl.ds(baset, TAIL)])


_gather_call = pl.kernel(
    _gather_body,
    out_type=jax.ShapeDtypeStruct((E, D), jnp.float32),
    mesh=_SC_MESH,
    scratch_types=[
        pltpu.VMEM((EPT,), jnp.int32),
        pltpu.VMEM((EPT,), jnp.int32),
        pltpu.VMEM((2, CH, D), jnp.float32),
        pltpu.VMEM((2, CH, D), jnp.float32),
        pltpu.VMEM((2, CH, D), jnp.float32),
        pltpu.SemaphoreType.DMA((2,)),
        pltpu.SemaphoreType.DMA((2,)),
        pltpu.SemaphoreType.DMA,
    ],
)


def _scatter_body(e_hbm, dst_hbm, zero_hbm, out_hbm,
                  didx, tidx, rows, accum, si, ss, st):
  c = lax.axis_index("c")
  s = lax.axis_index("s")
  wid = s * NC + c
  tb = pl.multiple_of(wid * EPT, 8)
  pltpu.sync_copy(zero_hbm.at[pl.ds(s * NPS, NPS)],
                  accum.at[pl.ds(s * NPS, NPS)])
  plsc.subcore_barrier()

  def fire_in(j, b):
    base = pl.multiple_of(tb + j * CH, 8)
    pltpu.async_copy(dst_hbm.at[pl.ds(base, CH)], didx.at[b], si.at[b])
    pltpu.async_copy(e_hbm.at[pl.ds(base, CH)], rows.at[b], si.at[b])

  def wait_in(j, b):
    base = pl.multiple_of(tb + j * CH, 8)
    pltpu.make_async_copy(dst_hbm.at[pl.ds(base, CH)], didx.at[b],
                          si.at[b]).wait()
    pltpu.make_async_copy(e_hbm.at[pl.ds(base, CH)], rows.at[b],
                          si.at[b]).wait()

  def drain_sc(b):
    pltpu.make_async_copy(rows.at[b], accum.at[didx.at[b]], ss.at[b]).wait()

  fire_in(0, 0)

  def body(j, carry):
    nb = (j + 1) % 2
    b = j % 2

    @pl.when(j + 1 < NCT)
    def _pref():
      @pl.when(j >= 1)
      def _():
        drain_sc(nb)
      fire_in(j + 1, nb)

    wait_in(j, b)
    pltpu.async_copy(rows.at[b], accum.at[didx.at[b]], ss.at[b], add=True)
    return carry

  lax.fori_loop(0, NCT, body, 0)
  baset = pl.multiple_of(tb + NCT * CH, 8)
  drain_sc(0)
  drain_sc(1)
  pltpu.sync_copy(dst_hbm.at[pl.ds(baset, TAIL)], tidx)
  pltpu.sync_copy(e_hbm.at[pl.ds(baset, TAIL)], rows.at[0, pl.ds(0, TAIL)])
  pltpu.sync_copy(rows.at[0, pl.ds(0, TAIL)], accum.at[tidx], add=True)
  plsc.subcore_barrier()
  pltpu.sync_copy(accum.at[pl.ds(s * NPS, NPS)],
                  out_hbm.at[c, pl.ds(s * NPS, NPS)])


_scatter_call = pl.kernel(
    _scatter_body,
    out_type=jax.ShapeDtypeStruct((NC, NPAD, D), jnp.float32),
    mesh=_SC_MESH,
    scratch_types=[
        pltpu.VMEM((2, CH), jnp.int32),
        pltpu.VMEM((TAIL,), jnp.int32),
        pltpu.VMEM((2, CH, D), jnp.float32),
        pltpu.VMEM_SHARED((NPAD, D), jnp.float32),
        pltpu.SemaphoreType.DMA((2,)),
        pltpu.SemaphoreType.DMA((2,)),
        pltpu.SemaphoreType.DMA,
    ],
)


def kernel(efeat, nfeat, edge_index, edge_w1, edge_b1, edge_w2, edge_b2,
           edge_ln_s, edge_ln_b, node_w1, node_b1, node_w2, node_b2,
           node_ln_s, node_ln_b):
  src = edge_index[0]
  dst = edge_index[1]
  zero = jnp.zeros((NPAD, D), jnp.float32)
  e, n = efeat, nfeat
  a_s, a_d = _pre_call(n, edge_w1, edge_b1, 0)
  for i in range(L):
    g = _gather_call(a_s, a_d, src, dst)
    e = _edge_call(e, g, edge_w1, edge_w2, edge_b2, edge_ln_s, edge_ln_b, i)
    parts = _scatter_call(e, dst, zero)
    if i + 1 < L:
      n, a_s, a_d = _node_pre_call(parts, n, node_w1, node_b1, node_w2,
                                   node_b2, node_ln_s, node_ln_b, i,
                                   edge_w1, edge_b1)
    else:
      n = _node_call(parts, n, node_w1, node_b1, node_w2, node_b2,
                     node_ln_s, node_ln_b, i)
  return (e, n)
